# baseline (speedup 1.0000x reference)
import jax
import jax.numpy as jnp
from jax.experimental import pallas as pl

_QT = 512


def _body(p2_ref, yy_ref, p1_ref, xx_ref, x1_ref, w1_ref, b1_ref, out_ref):
    k = p1_ref.shape[0]
    cross = jax.lax.dot_general(
        p1_ref[...], p2_ref[...], (((1,), (1,)), ((), ())),
        preferred_element_type=jnp.float32)
    d2 = (xx_ref[...] + yy_ref[...]) - 2.0 * cross
    d2 = jnp.maximum(d2, 0.0)

    big = jnp.float32(3e38)
    m = jnp.min(d2, axis=0, keepdims=True)
    for _ in range(7):
        m = jnp.min(jnp.where(d2 <= m, big, d2), axis=0, keepdims=True)

    w = jnp.where(d2 <= m, 1.0 / jnp.maximum(d2, 1e-16), 0.0)
    den = jnp.sum(w, axis=0, keepdims=True)
    w = w * (1.0 / den)

    h1 = jax.lax.dot_general(
        x1_ref[...], w1_ref[...], (((1,), (1,)), ((), ())),
        preferred_element_type=jnp.float32) + b1_ref[...]

    out_ref[...] = jax.lax.dot_general(
        w, h1, (((0,), (0,)), ((), ())),
        preferred_element_type=jnp.float32,
        precision=jax.lax.Precision.HIGHEST)


def kernel(x1, pos1, x2, pos2, batch1, batch2, W1, b1, W2, b2):
    n1, in_c = x1.shape
    n2 = pos2.shape[0]
    out_c = W1.shape[0]
    nb = 4
    k = n1 // nb
    qt = _QT
    tpb = (n2 // nb) // qt

    p1pad = jnp.pad(pos1, ((0, 0), (0, 5)))
    p2pad = jnp.pad(pos2, ((0, 0), (0, 5)))
    xx1 = jnp.sum(pos1 * pos1, axis=1, keepdims=True)
    yy2t = jnp.sum(pos2 * pos2, axis=1)[None, :]
    b1_2d = b1.reshape(1, out_c)

    out = pl.pallas_call(
        _body,
        grid=(n2 // qt,),
        in_specs=[
            pl.BlockSpec((qt, 8), lambda i: (i, 0)),
            pl.BlockSpec((1, qt), lambda i: (0, i)),
            pl.BlockSpec((k, 8), lambda i: (i // tpb, 0)),
            pl.BlockSpec((k, 1), lambda i: (i // tpb, 0)),
            pl.BlockSpec((k, in_c), lambda i: (i // tpb, 0)),
            pl.BlockSpec((out_c, in_c), lambda i: (0, 0)),
            pl.BlockSpec((1, out_c), lambda i: (0, 0)),
        ],
        out_specs=pl.BlockSpec((qt, out_c), lambda i: (i, 0)),
        out_shape=jax.ShapeDtypeStruct((n2, out_c), jnp.float32),
    )(p2pad, yy2t, p1pad, xx1, x1, W1, b1_2d)
    return out


# speedup vs baseline: 1.3353x; 1.3353x over previous
import jax
import jax.numpy as jnp
from jax.experimental import pallas as pl

_QT = 512


def _body(p2_ref, yy_ref, p1_ref, xx_ref, x1_ref, w1_ref, b1_ref, out_ref):
    k = p1_ref.shape[0]
    cross = jax.lax.dot_general(
        p1_ref[...], p2_ref[...], (((1,), (1,)), ((), ())),
        preferred_element_type=jnp.float32)
    d2 = (xx_ref[...] + yy_ref[...]) - 2.0 * cross
    d2 = jnp.maximum(d2, 0.0)

    big = jnp.float32(3e38)
    m = jnp.min(d2, axis=0, keepdims=True)
    for _ in range(7):
        m = jnp.min(jnp.where(d2 <= m, big, d2), axis=0, keepdims=True)

    w = jnp.where(d2 <= m, 1.0 / jnp.maximum(d2, 1e-16), 0.0)
    den = jnp.sum(w, axis=0, keepdims=True)
    w = w * (1.0 / den)

    h1 = jax.lax.dot_general(
        x1_ref[...], w1_ref[...], (((1,), (1,)), ((), ())),
        preferred_element_type=jnp.float32) + b1_ref[...]

    out_ref[...] = jax.lax.dot_general(
        w, h1, (((0,), (0,)), ((), ())),
        preferred_element_type=jnp.float32)


def kernel(x1, pos1, x2, pos2, batch1, batch2, W1, b1, W2, b2):
    n1, in_c = x1.shape
    n2 = pos2.shape[0]
    out_c = W1.shape[0]
    nb = 4
    k = n1 // nb
    qt = _QT
    tpb = (n2 // nb) // qt

    p1pad = jnp.pad(pos1, ((0, 0), (0, 5)))
    p2pad = jnp.pad(pos2, ((0, 0), (0, 5)))
    xx1 = jnp.sum(pos1 * pos1, axis=1, keepdims=True)
    yy2t = jnp.sum(pos2 * pos2, axis=1)[None, :]
    b1_2d = b1.reshape(1, out_c)

    out = pl.pallas_call(
        _body,
        grid=(n2 // qt,),
        in_specs=[
            pl.BlockSpec((qt, 8), lambda i: (i, 0)),
            pl.BlockSpec((1, qt), lambda i: (0, i)),
            pl.BlockSpec((k, 8), lambda i: (i // tpb, 0)),
            pl.BlockSpec((k, 1), lambda i: (i // tpb, 0)),
            pl.BlockSpec((k, in_c), lambda i: (i // tpb, 0)),
            pl.BlockSpec((out_c, in_c), lambda i: (0, 0)),
            pl.BlockSpec((1, out_c), lambda i: (0, 0)),
        ],
        out_specs=pl.BlockSpec((qt, out_c), lambda i: (i, 0)),
        out_shape=jax.ShapeDtypeStruct((n2, out_c), jnp.float32),
    )(p2pad, yy2t, p1pad, xx1, x1, W1, b1_2d)
    return out
